# baseline (speedup 1.0000x reference)
import jax
import jax.numpy as jnp
from jax import lax
from jax.experimental import pallas as pl
from jax.experimental.pallas import tpu as pltpu
from jax.experimental.pallas import tpu_sc as plsc

N_NODES = 10000
N_EDGES = 320000
NFEAT = 128
NHID = 128
NCLASS = 64

NC = 2
NS = 16
NW = NC * NS

N_PAD = 10240
ROWS_PER_TILE = N_PAD // NS
BLK = 128
E_PAD = 323584
EDGES_PER_W = E_PAD // NW
BLOCKS_PER_W = EDGES_PER_W // BLK


def _make_agg(d_feat: int, with_deg: bool):
    mesh = plsc.VectorSubcoreMesh(core_axis_name="c", subcore_axis_name="s")

    nbuf = 1 if d_feat > 64 else 2
    out_type = [jax.ShapeDtypeStruct((NC, N_NODES, d_feat), jnp.float32)]
    scratch = [
        pltpu.VMEM((BLOCKS_PER_W, BLK), jnp.int32),
        pltpu.VMEM((BLOCKS_PER_W, BLK), jnp.int32),
        [pltpu.VMEM((BLK, d_feat), jnp.float32) for _ in range(nbuf)],
        [pltpu.SemaphoreType.DMA for _ in range(nbuf)],
        [pltpu.SemaphoreType.DMA for _ in range(nbuf)],
        pltpu.VMEM_SHARED((N_PAD, d_feat), jnp.float32),
    ]
    if with_deg:
        out_type.append(jax.ShapeDtypeStruct((NW, N_PAD), jnp.float32))
        scratch.append(pltpu.VMEM((N_PAD,), jnp.float32))

    def body(feat_hbm, src_hbm, dst_hbm, zeros_hbm, *rest):
        if with_deg:
            (zerosn_hbm, part_hbm, degh_hbm,
             src_v, dst_v, rows, gsem, ssem, acc_sh, hist_v) = rest
        else:
            part_hbm, src_v, dst_v, rows, gsem, ssem, acc_sh = rest
        c = lax.axis_index("c")
        s = lax.axis_index("s")
        w = c * NS + s

        pltpu.sync_copy(zeros_hbm,
                        acc_sh.at[pl.ds(s * ROWS_PER_TILE, ROWS_PER_TILE)])
        pltpu.sync_copy(src_hbm.at[w], src_v)
        pltpu.sync_copy(dst_hbm.at[w], dst_v)
        if with_deg:
            pltpu.sync_copy(zerosn_hbm, hist_v)
        plsc.subcore_barrier()

        ones16 = jnp.full((16,), 1.0, jnp.float32)

        def hist_upd(i):
            if with_deg:
                for j in range(BLK // 16):
                    dv = dst_v[i, pl.ds(j * 16, 16)]
                    plsc.addupdate_scatter(hist_v, [dv], ones16)

        def gather(i, b):
            return pltpu.async_copy(feat_hbm.at[src_v.at[i]], rows[b], gsem[b])

        def scatter(i, b):
            return pltpu.async_copy(rows[b], acc_sh.at[dst_v.at[i]], ssem[b],
                                    add=True)

        if nbuf == 1:
            @pl.loop(0, BLOCKS_PER_W)
            def _(i):
                gather(i, 0).wait()
                sd = scatter(i, 0)
                hist_upd(i)
                sd.wait()
        else:
            def wait_gather(b):
                pltpu.make_async_copy(
                    feat_hbm.at[src_v.at[0]], rows[b], gsem[b]).wait()

            def wait_scatter(b):
                pltpu.make_async_copy(
                    rows[b], acc_sh.at[dst_v.at[0]], ssem[b]).wait()

            @pl.loop(0, BLOCKS_PER_W - 1, step=2)
            def _(i):
                g0 = gather(i, 0)
                g1 = gather(i + 1, 1)
                g0.wait()
                s0 = scatter(i, 0)
                hist_upd(i)
                g1.wait()
                s0.wait()
                s1 = scatter(i + 1, 1)
                hist_upd(i + 1)
                s1.wait()

            if BLOCKS_PER_W % 2:
                gather(BLOCKS_PER_W - 1, 0).wait()
                sd = scatter(BLOCKS_PER_W - 1, 0)
                hist_upd(BLOCKS_PER_W - 1)
                sd.wait()

        plsc.subcore_barrier()

        r0 = s * ROWS_PER_TILE
        tail = N_NODES - (NS - 1) * ROWS_PER_TILE
        if with_deg:
            pltpu.sync_copy(hist_v, degh_hbm.at[w])

        @pl.when(s < NS - 1)
        def _():
            pltpu.sync_copy(acc_sh.at[pl.ds(r0, ROWS_PER_TILE)],
                            part_hbm.at[c, pl.ds(r0, ROWS_PER_TILE)])

        @pl.when(s == NS - 1)
        def _():
            pltpu.sync_copy(acc_sh.at[pl.ds((NS - 1) * ROWS_PER_TILE, tail)],
                            part_hbm.at[c, pl.ds((NS - 1) * ROWS_PER_TILE, tail)])

    return pl.kernel(body, out_type=tuple(out_type), mesh=mesh,
                     scratch_types=scratch,
                     compiler_params=pltpu.CompilerParams(
                         use_tc_tiling_on_sc=False,
                         needs_layout_passes=False))


_agg1 = _make_agg(NFEAT, with_deg=True)
_agg2 = _make_agg(NCLASS, with_deg=False)

ROW_BLK = 2048


def _layer1_body(p_ref, degh_ref, w1t_ref, b1_ref, w2t_ref, z_ref):
    s = p_ref[0] + p_ref[1]
    deg = jnp.sum(degh_ref[...], axis=0)[:, None]
    mean = s / (deg + 1e-6)
    h = jnp.maximum(
        jnp.dot(mean, w1t_ref[...], preferred_element_type=jnp.float32)
        + b1_ref[...], 0.0)
    z_ref[...] = jnp.dot(h, w2t_ref[...], preferred_element_type=jnp.float32)


def _layer2_body(q_ref, degh_ref, b2_ref, out_ref):
    s = q_ref[0] + q_ref[1]
    deg = jnp.sum(degh_ref[...], axis=0)[:, None]
    t = s / (deg + 1e-6) + b2_ref[...]
    m = jnp.max(t, axis=1, keepdims=True)
    ls = jnp.log(jnp.sum(jnp.exp(t - m), axis=1, keepdims=True)) + m
    out_ref[...] = t - ls


def _tc_layer1(p, degh, w1t, b1, w2t):
    return pl.pallas_call(
        _layer1_body,
        grid=(pl.cdiv(N_NODES, ROW_BLK),),
        in_specs=[
            pl.BlockSpec((NC, ROW_BLK, NFEAT), lambda i: (0, i, 0)),
            pl.BlockSpec((NW, ROW_BLK), lambda i: (0, i)),
            pl.BlockSpec((NFEAT, NHID), lambda i: (0, 0)),
            pl.BlockSpec((1, NHID), lambda i: (0, 0)),
            pl.BlockSpec((NHID, NCLASS), lambda i: (0, 0)),
        ],
        out_specs=pl.BlockSpec((ROW_BLK, NCLASS), lambda i: (i, 0)),
        out_shape=jax.ShapeDtypeStruct((N_NODES, NCLASS), jnp.float32),
    )(p, degh, w1t, b1, w2t)


def _tc_layer2(q, degh, b2):
    return pl.pallas_call(
        _layer2_body,
        grid=(pl.cdiv(N_NODES, ROW_BLK),),
        in_specs=[
            pl.BlockSpec((NC, ROW_BLK, NCLASS), lambda i: (0, i, 0)),
            pl.BlockSpec((NW, ROW_BLK), lambda i: (0, i)),
            pl.BlockSpec((1, NCLASS), lambda i: (0, 0)),
        ],
        out_specs=pl.BlockSpec((ROW_BLK, NCLASS), lambda i: (i, 0)),
        out_shape=jax.ShapeDtypeStruct((N_NODES, NCLASS), jnp.float32),
    )(q, degh, b2)


def kernel(x, edge_index, W1, b1, W2, b2):
    n_extra = E_PAD - N_EDGES
    pad_src = (jnp.arange(n_extra, dtype=jnp.int32) * 13) % N_NODES
    pad_dst = N_NODES + (jnp.arange(n_extra, dtype=jnp.int32) % (N_PAD - N_NODES))
    src = jnp.concatenate(
        [edge_index[0].astype(jnp.int32), pad_src]).reshape(
            NW, BLOCKS_PER_W, BLK)
    dst = jnp.concatenate(
        [edge_index[1].astype(jnp.int32), pad_dst]).reshape(
            NW, BLOCKS_PER_W, BLK)

    zeros128 = jnp.zeros((ROWS_PER_TILE, NFEAT), jnp.float32)
    zeros64 = jnp.zeros((ROWS_PER_TILE, NCLASS), jnp.float32)
    zerosn = jnp.zeros((N_PAD,), jnp.float32)

    p, degh = _agg1(x, src, dst, zeros128, zerosn)
    z = _tc_layer1(p, degh, W1.T, b1.reshape(1, NHID), W2.T)
    (q,) = _agg2(z, src, dst, zeros64)
    return _tc_layer2(q, degh, b2.reshape(1, NCLASS))


# speedup vs baseline: 1.1696x; 1.1696x over previous
import jax
import jax.numpy as jnp
from jax import lax
from jax.experimental import pallas as pl
from jax.experimental.pallas import tpu as pltpu
from jax.experimental.pallas import tpu_sc as plsc

N_NODES = 10000
N_EDGES = 320000
NFEAT = 128
NHID = 128
NCLASS = 64

NC = 2
NS = 16
NW = NC * NS

N_PAD = 10240
ROWS_PER_TILE = N_PAD // NS
E_PAD = 327680
EDGES_PER_W = E_PAD // NW
KBODY = 8


def _make_agg(d_feat: int, blk: int, with_deg: bool):
    mesh = plsc.VectorSubcoreMesh(core_axis_name="c", subcore_axis_name="s")

    blocks = EDGES_PER_W // blk
    assert blocks % KBODY == 0
    nbuf = 2
    out_type = [jax.ShapeDtypeStruct((NC, N_NODES, d_feat), jnp.float32)]
    scratch = [
        pltpu.VMEM((blocks, blk), jnp.int32),
        pltpu.VMEM((blocks, blk), jnp.int32),
        [pltpu.VMEM((blk, d_feat), jnp.float32) for _ in range(nbuf)],
        [pltpu.SemaphoreType.DMA for _ in range(nbuf)],
        [pltpu.SemaphoreType.DMA for _ in range(nbuf)],
        pltpu.VMEM_SHARED((N_PAD, d_feat), jnp.float32),
    ]
    if with_deg:
        out_type.append(jax.ShapeDtypeStruct((NW, N_PAD), jnp.float32))
        scratch.append(pltpu.VMEM((N_PAD,), jnp.float32))

    def body(feat_hbm, src_hbm, dst_hbm, zeros_hbm, *rest):
        if with_deg:
            (zerosn_hbm, part_hbm, degh_hbm,
             src_v, dst_v, rows, gsem, ssem, acc_sh, hist_v) = rest
        else:
            part_hbm, src_v, dst_v, rows, gsem, ssem, acc_sh = rest
        c = lax.axis_index("c")
        s = lax.axis_index("s")
        w = c * NS + s

        pltpu.sync_copy(zeros_hbm,
                        acc_sh.at[pl.ds(s * ROWS_PER_TILE, ROWS_PER_TILE)])
        pltpu.sync_copy(src_hbm.at[w], src_v)
        pltpu.sync_copy(dst_hbm.at[w], dst_v)
        if with_deg:
            pltpu.sync_copy(zerosn_hbm, hist_v)
        plsc.subcore_barrier()

        ones16 = jnp.full((16,), 1.0, jnp.float32)

        def hist_upd(i):
            if with_deg:
                for j in range(blk // 16):
                    dv = dst_v[i, pl.ds(j * 16, 16)]
                    plsc.addupdate_scatter(hist_v, [dv], ones16)

        def gather(i, b):
            return pltpu.async_copy(feat_hbm.at[src_v.at[i]], rows[b], gsem[b])

        def scatter(i, b):
            return pltpu.async_copy(rows[b], acc_sh.at[dst_v.at[i]], ssem[b],
                                    add=True)

        @pl.loop(0, blocks, step=KBODY)
        def _(i):
            gd = {0: gather(i, 0), 1: gather(i + 1, 1)}
            sd = {}
            for k in range(KBODY):
                b = k % 2
                gd[k].wait()
                sd[k] = scatter(i + k, b)
                hist_upd(i + k)
                if k + 2 < KBODY:
                    sd[k].wait()
                    gd[k + 2] = gather(i + k + 2, b)
            sd[KBODY - 2].wait()
            sd[KBODY - 1].wait()

        plsc.subcore_barrier()

        r0 = s * ROWS_PER_TILE
        tail = N_NODES - (NS - 1) * ROWS_PER_TILE
        if with_deg:
            pltpu.sync_copy(hist_v, degh_hbm.at[w])

        @pl.when(s < NS - 1)
        def _():
            pltpu.sync_copy(acc_sh.at[pl.ds(r0, ROWS_PER_TILE)],
                            part_hbm.at[c, pl.ds(r0, ROWS_PER_TILE)])

        @pl.when(s == NS - 1)
        def _():
            pltpu.sync_copy(acc_sh.at[pl.ds((NS - 1) * ROWS_PER_TILE, tail)],
                            part_hbm.at[c, pl.ds((NS - 1) * ROWS_PER_TILE, tail)])

    return pl.kernel(body, out_type=tuple(out_type), mesh=mesh,
                     scratch_types=scratch,
                     compiler_params=pltpu.CompilerParams(
                         use_tc_tiling_on_sc=False,
                         needs_layout_passes=False))


_agg1 = _make_agg(NFEAT, blk=64, with_deg=True)
_agg2 = _make_agg(NCLASS, blk=128, with_deg=False)

ROW_BLK = 2048


def _layer1_body(p_ref, degh_ref, w1t_ref, b1_ref, w2t_ref, z_ref):
    s = p_ref[0] + p_ref[1]
    deg = jnp.sum(degh_ref[...], axis=0)[:, None]
    mean = s / (deg + 1e-6)
    h = jnp.maximum(
        jnp.dot(mean, w1t_ref[...], preferred_element_type=jnp.float32)
        + b1_ref[...], 0.0)
    z_ref[...] = jnp.dot(h, w2t_ref[...], preferred_element_type=jnp.float32)


def _layer2_body(q_ref, degh_ref, b2_ref, out_ref):
    s = q_ref[0] + q_ref[1]
    deg = jnp.sum(degh_ref[...], axis=0)[:, None]
    t = s / (deg + 1e-6) + b2_ref[...]
    m = jnp.max(t, axis=1, keepdims=True)
    ls = jnp.log(jnp.sum(jnp.exp(t - m), axis=1, keepdims=True)) + m
    out_ref[...] = t - ls


def _tc_layer1(p, degh, w1t, b1, w2t):
    return pl.pallas_call(
        _layer1_body,
        grid=(pl.cdiv(N_NODES, ROW_BLK),),
        in_specs=[
            pl.BlockSpec((NC, ROW_BLK, NFEAT), lambda i: (0, i, 0)),
            pl.BlockSpec((NW, ROW_BLK), lambda i: (0, i)),
            pl.BlockSpec((NFEAT, NHID), lambda i: (0, 0)),
            pl.BlockSpec((1, NHID), lambda i: (0, 0)),
            pl.BlockSpec((NHID, NCLASS), lambda i: (0, 0)),
        ],
        out_specs=pl.BlockSpec((ROW_BLK, NCLASS), lambda i: (i, 0)),
        out_shape=jax.ShapeDtypeStruct((N_NODES, NCLASS), jnp.float32),
    )(p, degh, w1t, b1, w2t)


def _tc_layer2(q, degh, b2):
    return pl.pallas_call(
        _layer2_body,
        grid=(pl.cdiv(N_NODES, ROW_BLK),),
        in_specs=[
            pl.BlockSpec((NC, ROW_BLK, NCLASS), lambda i: (0, i, 0)),
            pl.BlockSpec((NW, ROW_BLK), lambda i: (0, i)),
            pl.BlockSpec((1, NCLASS), lambda i: (0, 0)),
        ],
        out_specs=pl.BlockSpec((ROW_BLK, NCLASS), lambda i: (i, 0)),
        out_shape=jax.ShapeDtypeStruct((N_NODES, NCLASS), jnp.float32),
    )(q, degh, b2)


def kernel(x, edge_index, W1, b1, W2, b2):
    n_extra = E_PAD - N_EDGES
    pad_src = (jnp.arange(n_extra, dtype=jnp.int32) * 13) % N_NODES
    pad_dst = N_NODES + (jnp.arange(n_extra, dtype=jnp.int32) % (N_PAD - N_NODES))
    src = jnp.concatenate([edge_index[0].astype(jnp.int32), pad_src])
    dst = jnp.concatenate([edge_index[1].astype(jnp.int32), pad_dst])
    src64 = src.reshape(NW, EDGES_PER_W // 64, 64)
    dst64 = dst.reshape(NW, EDGES_PER_W // 64, 64)
    src128 = src.reshape(NW, EDGES_PER_W // 128, 128)
    dst128 = dst.reshape(NW, EDGES_PER_W // 128, 128)

    zeros128 = jnp.zeros((ROWS_PER_TILE, NFEAT), jnp.float32)
    zeros64 = jnp.zeros((ROWS_PER_TILE, NCLASS), jnp.float32)
    zerosn = jnp.zeros((N_PAD,), jnp.float32)

    p, degh = _agg1(x, src64, dst64, zeros128, zerosn)
    z = _tc_layer1(p, degh, W1.T, b1.reshape(1, NHID), W2.T)
    (q,) = _agg2(z, src128, dst128, zeros64)
    return _tc_layer2(q, degh, b2.reshape(1, NCLASS))


# speedup vs baseline: 1.2432x; 1.0629x over previous
import jax
import jax.numpy as jnp
from jax import lax
from jax.experimental import pallas as pl
from jax.experimental.pallas import tpu as pltpu
from jax.experimental.pallas import tpu_sc as plsc

N_NODES = 10000
N_EDGES = 320000
NFEAT = 128
NHID = 128
NCLASS = 64

NC = 2
NS = 16
NW = NC * NS

N_PAD = 10240
ROWS_PER_TILE = N_PAD // NS
E_PAD = 327680
EDGES_PER_W = E_PAD // NW
KBODY = 8


def _make_agg(d_feat: int, blk: int, with_deg: bool):
    mesh = plsc.VectorSubcoreMesh(core_axis_name="c", subcore_axis_name="s")

    blocks = EDGES_PER_W // blk
    assert blocks % KBODY == 0
    nbuf = 2
    out_type = [jax.ShapeDtypeStruct((NC, N_NODES, d_feat), jnp.float32)]
    scratch = [
        pltpu.VMEM((blocks, blk), jnp.int32),
        pltpu.VMEM((blocks, blk), jnp.int32),
        [pltpu.VMEM((blk, d_feat), jnp.float32) for _ in range(nbuf)],
        [pltpu.SemaphoreType.DMA for _ in range(nbuf)],
        [pltpu.SemaphoreType.DMA for _ in range(nbuf)],
        pltpu.VMEM_SHARED((N_PAD, d_feat), jnp.float32),
    ]
    if with_deg:
        out_type.append(jax.ShapeDtypeStruct((NW, N_PAD), jnp.float32))
        scratch.append(pltpu.VMEM((N_PAD,), jnp.float32))

    def body(feat_hbm, src_hbm, dst_hbm, zeros_hbm, *rest):
        if with_deg:
            (zerosn_hbm, part_hbm, degh_hbm,
             src_v, dst_v, rows, gsem, ssem, acc_sh, hist_v) = rest
        else:
            part_hbm, src_v, dst_v, rows, gsem, ssem, acc_sh = rest
        c = lax.axis_index("c")
        s = lax.axis_index("s")
        w = c * NS + s

        pltpu.sync_copy(zeros_hbm,
                        acc_sh.at[pl.ds(s * ROWS_PER_TILE, ROWS_PER_TILE)])
        pltpu.sync_copy(src_hbm.at[w], src_v)
        pltpu.sync_copy(dst_hbm.at[w], dst_v)
        if with_deg:
            pltpu.sync_copy(zerosn_hbm, hist_v)
        plsc.subcore_barrier()

        ones16 = jnp.full((16,), 1.0, jnp.float32)

        def hist_upd(i):
            if with_deg:
                for j in range(blk // 16):
                    dv = dst_v[i, pl.ds(j * 16, 16)]
                    plsc.addupdate_scatter(hist_v, [dv], ones16)

        def gather(i, b):
            return pltpu.async_copy(feat_hbm.at[src_v.at[i]], rows[b], gsem[b])

        def scatter(i, b):
            return pltpu.async_copy(rows[b], acc_sh.at[dst_v.at[i]], ssem[b],
                                    add=True)

        gather(0, 0)
        gather(1, 1)

        @pl.loop(0, blocks, step=KBODY)
        def _(i):
            sd = {}
            for k in range(KBODY):
                b = k % 2
                pltpu.make_async_copy(
                    feat_hbm.at[src_v.at[i + k]], rows[b], gsem[b]).wait()
                sd[k] = scatter(i + k, b)
                hist_upd(i + k)
                sd[k].wait()
                nxt = i + k + 2

                @pl.when(nxt < blocks)
                def _():
                    gather(nxt, b)

        plsc.subcore_barrier()

        r0 = s * ROWS_PER_TILE
        tail = N_NODES - (NS - 1) * ROWS_PER_TILE
        if with_deg:
            pltpu.sync_copy(hist_v, degh_hbm.at[w])

        @pl.when(s < NS - 1)
        def _():
            pltpu.sync_copy(acc_sh.at[pl.ds(r0, ROWS_PER_TILE)],
                            part_hbm.at[c, pl.ds(r0, ROWS_PER_TILE)])

        @pl.when(s == NS - 1)
        def _():
            pltpu.sync_copy(acc_sh.at[pl.ds((NS - 1) * ROWS_PER_TILE, tail)],
                            part_hbm.at[c, pl.ds((NS - 1) * ROWS_PER_TILE, tail)])

    return pl.kernel(body, out_type=tuple(out_type), mesh=mesh,
                     scratch_types=scratch,
                     compiler_params=pltpu.CompilerParams(
                         use_tc_tiling_on_sc=False,
                         needs_layout_passes=False))


_agg1 = _make_agg(NFEAT, blk=64, with_deg=True)
_agg2 = _make_agg(NCLASS, blk=128, with_deg=False)

ROW_BLK = 2048


def _layer1_body(p_ref, degh_ref, w1t_ref, b1_ref, w2t_ref, z_ref):
    s = p_ref[0] + p_ref[1]
    deg = jnp.sum(degh_ref[...], axis=0)[:, None]
    mean = s / (deg + 1e-6)
    h = jnp.maximum(
        jnp.dot(mean, w1t_ref[...], preferred_element_type=jnp.float32)
        + b1_ref[...], 0.0)
    z_ref[...] = jnp.dot(h, w2t_ref[...], preferred_element_type=jnp.float32)


def _layer2_body(q_ref, degh_ref, b2_ref, out_ref):
    s = q_ref[0] + q_ref[1]
    deg = jnp.sum(degh_ref[...], axis=0)[:, None]
    t = s / (deg + 1e-6) + b2_ref[...]
    m = jnp.max(t, axis=1, keepdims=True)
    ls = jnp.log(jnp.sum(jnp.exp(t - m), axis=1, keepdims=True)) + m
    out_ref[...] = t - ls


def _tc_layer1(p, degh, w1t, b1, w2t):
    return pl.pallas_call(
        _layer1_body,
        grid=(pl.cdiv(N_NODES, ROW_BLK),),
        in_specs=[
            pl.BlockSpec((NC, ROW_BLK, NFEAT), lambda i: (0, i, 0)),
            pl.BlockSpec((NW, ROW_BLK), lambda i: (0, i)),
            pl.BlockSpec((NFEAT, NHID), lambda i: (0, 0)),
            pl.BlockSpec((1, NHID), lambda i: (0, 0)),
            pl.BlockSpec((NHID, NCLASS), lambda i: (0, 0)),
        ],
        out_specs=pl.BlockSpec((ROW_BLK, NCLASS), lambda i: (i, 0)),
        out_shape=jax.ShapeDtypeStruct((N_NODES, NCLASS), jnp.float32),
    )(p, degh, w1t, b1, w2t)


def _tc_layer2(q, degh, b2):
    return pl.pallas_call(
        _layer2_body,
        grid=(pl.cdiv(N_NODES, ROW_BLK),),
        in_specs=[
            pl.BlockSpec((NC, ROW_BLK, NCLASS), lambda i: (0, i, 0)),
            pl.BlockSpec((NW, ROW_BLK), lambda i: (0, i)),
            pl.BlockSpec((1, NCLASS), lambda i: (0, 0)),
        ],
        out_specs=pl.BlockSpec((ROW_BLK, NCLASS), lambda i: (i, 0)),
        out_shape=jax.ShapeDtypeStruct((N_NODES, NCLASS), jnp.float32),
    )(q, degh, b2)


def kernel(x, edge_index, W1, b1, W2, b2):
    n_extra = E_PAD - N_EDGES
    pad_src = (jnp.arange(n_extra, dtype=jnp.int32) * 13) % N_NODES
    pad_dst = N_NODES + (jnp.arange(n_extra, dtype=jnp.int32) % (N_PAD - N_NODES))
    src = jnp.concatenate([edge_index[0].astype(jnp.int32), pad_src])
    dst = jnp.concatenate([edge_index[1].astype(jnp.int32), pad_dst])
    src64 = src.reshape(NW, EDGES_PER_W // 64, 64)
    dst64 = dst.reshape(NW, EDGES_PER_W // 64, 64)
    src128 = src.reshape(NW, EDGES_PER_W // 128, 128)
    dst128 = dst.reshape(NW, EDGES_PER_W // 128, 128)

    zeros128 = jnp.zeros((ROWS_PER_TILE, NFEAT), jnp.float32)
    zeros64 = jnp.zeros((ROWS_PER_TILE, NCLASS), jnp.float32)
    zerosn = jnp.zeros((N_PAD,), jnp.float32)

    p, degh = _agg1(x, src64, dst64, zeros128, zerosn)
    z = _tc_layer1(p, degh, W1.T, b1.reshape(1, NHID), W2.T)
    (q,) = _agg2(z, src128, dst128, zeros64)
    return _tc_layer2(q, degh, b2.reshape(1, NCLASS))
